# baseline (device time: 53926 ns/iter reference)
import jax
import jax.numpy as jnp
from jax import lax
from jax.experimental import pallas as pl
from jax.experimental.pallas import tpu as pltpu

N_DEV = 4
B, SQ, HQ, DH = 2, 256, 4, 64
SKV = N_DEV * SQ
BLK = 64

_MESH = pl.DeviceIdType.MESH


def kernel(x, Wq, K_ext, V_ext, Wo):
    d_model = x.shape[-1]

    def body(x_ref, wq_ref, k_ref, v_ref, wo_ref, out_ref,
             kfull, vfull, send_sems, recv_sems):
        my = lax.axis_index("i")

        barrier = pltpu.get_barrier_semaphore()
        for d in (1, 2, 3):
            pl.semaphore_signal(
                barrier, inc=1,
                device_id=((my + d) % N_DEV,), device_id_type=_MESH,
            )
        pl.semaphore_wait(barrier, N_DEV - 1)

        sends = []
        for d in (1, 2, 3):
            tgt = (my + d) % N_DEV
            for t, (src, full) in enumerate(((k_ref, kfull), (v_ref, vfull))):
                c = pltpu.make_async_remote_copy(
                    src_ref=src,
                    dst_ref=full.at[my],
                    send_sem=send_sems.at[t * 3 + (d - 1)],
                    recv_sem=recv_sems.at[t * 3 + (d - 1)],
                    device_id=(tgt,),
                    device_id_type=_MESH,
                )
                c.start()
                sends.append(c)

        kfull[my] = k_ref[...]
        vfull[my] = v_ref[...]
        q = [
            jnp.dot(x_ref[b], wq_ref[...], preferred_element_type=jnp.float32)
            for b in range(B)
        ]

        for d in (1, 2, 3):
            origin = (my - d) % N_DEV
            for t, (src, full) in enumerate(((k_ref, kfull), (v_ref, vfull))):
                c = pltpu.make_async_remote_copy(
                    src_ref=src,
                    dst_ref=full.at[origin],
                    send_sem=send_sems.at[t * 3 + (d - 1)],
                    recv_sem=recv_sems.at[t * 3 + (d - 1)],
                    device_id=((my + d) % N_DEV,),
                    device_id_type=_MESH,
                )
                c.wait_recv()

        out_ref[...] = x_ref[...]

        for c in sends:
            c.wait_send()

    out_shape = jax.ShapeDtypeStruct((B, SQ, d_model), jnp.float32)
    return pl.pallas_call(
        body,
        out_shape=out_shape,
        in_specs=[pl.BlockSpec(memory_space=pltpu.VMEM)] * 5,
        out_specs=pl.BlockSpec(memory_space=pltpu.VMEM),
        scratch_shapes=[
            pltpu.VMEM((N_DEV, B, SQ, HQ, DH), jnp.float32),
            pltpu.VMEM((N_DEV, B, SQ, HQ, DH), jnp.float32),
            pltpu.SemaphoreType.DMA((6,)),
            pltpu.SemaphoreType.DMA((6,)),
        ],
        compiler_params=pltpu.CompilerParams(collective_id=0),
    )(x, Wq, K_ext, V_ext, Wo)


# device time: 29185 ns/iter; 1.8477x vs baseline; 1.8477x over previous
import jax
import jax.numpy as jnp
from jax import lax
from jax.experimental import pallas as pl
from jax.experimental.pallas import tpu as pltpu

N_DEV = 4
B, SQ, HQ, DH = 2, 256, 4, 64
BLK = 64
NB = SQ // BLK

_MESH = pl.DeviceIdType.MESH

R_FL0, R_FL1, R_FR0, R_FR1, R_DG0, R_DG1 = range(6)
S_TL0, S_TL1, S_TR0, S_TR1, S_FWL, S_FWR = range(6)


def kernel(x, Wq, K_ext, V_ext, Wo):
    d_model = x.shape[-1]

    def body(x_ref, wq_ref, k_ref, v_ref, wo_ref, out_ref,
             kv_send, kvfull, send_sems, recv_sems):
        my = lax.axis_index("i")
        left = (my - 1) % N_DEV
        right = (my + 1) % N_DEV
        diag = (my + 2) % N_DEV

        def copy(src, dst, s_sem, r_sem, dev):
            return pltpu.make_async_remote_copy(
                src_ref=src, dst_ref=dst,
                send_sem=send_sems.at[s_sem], recv_sem=recv_sems.at[r_sem],
                device_id=(dev,), device_id_type=_MESH,
            )

        kv_send[:, :, 0:HQ] = k_ref[...].astype(jnp.bfloat16)
        kv_send[:, :, HQ:2 * HQ] = v_ref[...].astype(jnp.bfloat16)

        barrier = pltpu.get_barrier_semaphore()
        for nbr in (left, right):
            pl.semaphore_signal(
                barrier, inc=1, device_id=(nbr,), device_id_type=_MESH,
            )
        pl.semaphore_wait(barrier, 2)

        sends = [
            copy(kv_send.at[0], kvfull.at[my, 0], S_TL0, R_FR0, left),
            copy(kv_send.at[1], kvfull.at[my, 1], S_TR1, R_FL1, right),
            copy(kv_send.at[1], kvfull.at[my, 1], S_TL1, R_FR1, left),
            copy(kv_send.at[0], kvfull.at[my, 0], S_TR0, R_FL0, right),
        ]
        for c in sends:
            c.start()

        wq_s = (wq_ref[...] * 0.125).astype(jnp.bfloat16)
        q = [
            jnp.dot(x_ref[b].astype(jnp.bfloat16), wq_s,
                    preferred_element_type=jnp.float32)
            for b in range(B)
        ]
        q_bh = [
            [q[b][:, h * DH:(h + 1) * DH].astype(jnp.bfloat16)
             for h in range(HQ)]
            for b in range(B)
        ]

        rr = lax.broadcasted_iota(jnp.int32, (SQ, SQ), 0)
        cc = lax.broadcasted_iota(jnp.int32, (SQ, SQ), 1)
        qb = my * NB + rr // BLK
        neg = jnp.float32(-1e9)
        zero = jnp.float32(0.0)

        def bias_for(j):
            kb = j * NB + cc // BLK
            mask = (qb == kb) | (kb == 0) | (((qb + kb) % 3) == 0)
            return jnp.where(mask, zero, neg)

        acc = [[None] * HQ for _ in range(B)]
        lsum = [[None] * HQ for _ in range(B)]

        def chunk(b, bias, kc_ref, first):
            for h in range(HQ):
                k_c = kc_ref[:, h, :]
                v_c = kc_ref[:, HQ + h, :]
                s = lax.dot_general(
                    q_bh[b][h], k_c, (((1,), (1,)), ((), ())),
                    preferred_element_type=jnp.float32,
                )
                w = jnp.exp(s + bias)
                part_l = jnp.sum(w, axis=1, keepdims=True)
                part = jnp.dot(w.astype(jnp.bfloat16), v_c,
                               preferred_element_type=jnp.float32)
                if first:
                    acc[b][h] = part
                    lsum[b][h] = part_l
                else:
                    acc[b][h] = acc[b][h] + part
                    lsum[b][h] = lsum[b][h] + part_l

        def wait(r_sem, j, b, dev):
            copy(kv_send.at[b], kvfull.at[j, b], S_TL0, r_sem, dev).wait_recv()

        bias_my = bias_for(my)
        chunk(0, bias_my, kv_send.at[0], first=True)

        wait(R_FL1, left, 1, left)
        fwd_r = copy(kvfull.at[left, 1], kvfull.at[left, 1], S_FWR, R_DG1,
                     right)
        fwd_r.start()
        wait(R_FR0, right, 0, right)
        fwd_l = copy(kvfull.at[right, 0], kvfull.at[right, 0], S_FWL, R_DG0,
                     left)
        fwd_l.start()
        sends += [fwd_r, fwd_l]

        chunk(1, bias_my, kv_send.at[1], first=True)

        bias_l = bias_for(left)
        chunk(1, bias_l, kvfull.at[left, 1], first=False)
        wait(R_FL0, left, 0, left)
        chunk(0, bias_l, kvfull.at[left, 0], first=False)

        bias_r = bias_for(right)
        chunk(0, bias_r, kvfull.at[right, 0], first=False)
        wait(R_FR1, right, 1, right)
        chunk(1, bias_r, kvfull.at[right, 1], first=False)

        bias_d = bias_for(diag)
        wait(R_DG0, diag, 0, right)
        chunk(0, bias_d, kvfull.at[diag, 0], first=False)
        wait(R_DG1, diag, 1, left)
        chunk(1, bias_d, kvfull.at[diag, 1], first=False)

        wo_b = wo_ref[...].astype(jnp.bfloat16)
        for b in range(B):
            ctx = jnp.concatenate(
                [acc[b][h] / lsum[b][h] for h in range(HQ)], axis=1
            )
            out_ref[b] = jnp.dot(ctx.astype(jnp.bfloat16), wo_b,
                                 preferred_element_type=jnp.float32)

        for c in sends:
            c.wait_send()

    out_shape = jax.ShapeDtypeStruct((B, SQ, d_model), jnp.float32)
    return pl.pallas_call(
        body,
        out_shape=out_shape,
        in_specs=[pl.BlockSpec(memory_space=pltpu.VMEM)] * 5,
        out_specs=pl.BlockSpec(memory_space=pltpu.VMEM),
        scratch_shapes=[
            pltpu.VMEM((B, SQ, 2 * HQ, DH), jnp.bfloat16),
            pltpu.VMEM((N_DEV, B, SQ, 2 * HQ, DH), jnp.bfloat16),
            pltpu.SemaphoreType.DMA((6,)),
            pltpu.SemaphoreType.DMA((6,)),
        ],
        compiler_params=pltpu.CompilerParams(collective_id=0),
    )(x, Wq, K_ext, V_ext, Wo)
